# baseline (speedup 1.0000x reference)
import functools

import jax
import jax.numpy as jnp
from jax import lax
from jax.experimental import pallas as pl
from jax.experimental.pallas import tpu as pltpu
from jax.experimental.pallas import tpu_sc as plsc

NC = 2
NS = 16
NW = NC * NS
CHUNK = 128
BROWS = 80


def _sc_hops(e, n_nodes, n_pad, d, trip):
  nblocks = n_pad // BROWS
  wblocks = n_nodes // BROWS
  epad = trip * NW * CHUNK
  mesh = plsc.VectorSubcoreMesh(core_axis_name="c", subcore_axis_name="s")

  @functools.partial(
      pl.kernel,
      out_type=jax.ShapeDtypeStruct((NC, n_nodes, d), jnp.float32),
      mesh=mesh,
      scratch_types=[
          pltpu.VMEM((CHUNK,), jnp.int32),
          pltpu.VMEM((CHUNK,), jnp.int32),
          pltpu.VMEM((CHUNK,), jnp.int32),
          pltpu.VMEM((CHUNK,), jnp.int32),
          pltpu.VMEM((CHUNK, d), jnp.float32),
          pltpu.VMEM((CHUNK, d), jnp.float32),
          pltpu.VMEM((8, d), jnp.float32),
          pltpu.VMEM_SHARED((n_pad, d), jnp.float32),
          pltpu.SemaphoreType.DMA,
          pltpu.SemaphoreType.DMA,
          pltpu.SemaphoreType.DMA,
          pltpu.SemaphoreType.DMA,
      ],
  )
  def sc_kernel(x0_hbm, x1_hbm, x2_hbm, ea_hbm, ei0_hbm, ei1_hbm, ei2_hbm,
                out_hbm, sv0, sv1, dv0, dv1, mv0, mv1, zero_v, acc,
                isem0, isem1, gsem0, gsem1):
    src_v = (sv0, sv1)
    dst_v = (dv0, dv1)
    msg_v = (mv0, mv1)
    isem = (isem0, isem1)
    gsem = (gsem0, gsem1)

    c = lax.axis_index("c")
    s = lax.axis_index("s")
    w = c * NS + s

    zvec = jnp.zeros((16,), jnp.float32)
    for k in range(d // 16):
      for r in range(8):
        zero_v[r, pl.ds(16 * k, 16)] = zvec

    def zero_body(j, carry):
      blk = s + j * NS
      for m in range(BROWS // 8):
        pltpu.sync_copy(zero_v, acc.at[pl.ds(blk * BROWS + m * 8, 8), :])
      return carry
    lax.fori_loop(0, (nblocks - s + NS - 1) // NS, zero_body, 0)

    plsc.subcore_barrier()

    def phase(ei_hbm, fire_load, wait_load, need_src):
      def fire_idx(i, b):
        base = (w + i * NW) * CHUNK
        if need_src:
          pltpu.async_copy(ei_hbm.at[pl.ds(base, CHUNK)], src_v[b], isem[b])
        pltpu.async_copy(ei_hbm.at[pl.ds(epad + base, CHUNK)], dst_v[b], isem[b])

      def wait_idx(b):
        if need_src:
          pltpu.make_async_copy(ei_hbm.at[pl.ds(0, CHUNK)], src_v[b], isem[b]).wait()
        pltpu.make_async_copy(ei_hbm.at[pl.ds(0, CHUNK)], dst_v[b], isem[b]).wait()

      def run_iter(i, b):
        nb = 1 - b
        wait_load(b)

        @pl.when(i + 1 < trip)
        def _():
          wait_idx(nb)
          fire_load(i + 1, nb)

        pltpu.sync_copy(msg_v[b], acc.at[dst_v[b]], add=True)

        @pl.when(i + 2 < trip)
        def _():
          fire_idx(i + 2, b)

      base0 = w * CHUNK
      if need_src:
        pltpu.sync_copy(ei_hbm.at[pl.ds(base0, CHUNK)], src_v[0])
      pltpu.sync_copy(ei_hbm.at[pl.ds(epad + base0, CHUNK)], dst_v[0])
      fire_load(0, 0)
      fire_idx(1, 1)

      def loop_body(j, carry):
        run_iter(2 * j, 0)
        run_iter(2 * j + 1, 1)
        return carry
      lax.fori_loop(0, trip // 2, loop_body, 0)

    def hop(x_hbm, ei_hbm):
      def fire_load(i, b):
        del i
        pltpu.async_copy(x_hbm.at[src_v[b]], msg_v[b], gsem[b])

      def wait_load(b):
        pltpu.make_async_copy(x_hbm.at[pl.ds(0, CHUNK), :], msg_v[b], gsem[b]).wait()

      phase(ei_hbm, fire_load, wait_load, need_src=True)

    def ea_phase():
      def fire_load(i, b):
        base = jnp.minimum((w + i * NW) * CHUNK, e - CHUNK)
        pltpu.async_copy(ea_hbm.at[pl.ds(base, CHUNK), :], msg_v[b], gsem[b])

      def wait_load(b):
        pltpu.make_async_copy(ea_hbm.at[pl.ds(0, CHUNK), :], msg_v[b], gsem[b]).wait()

      phase(ei0_hbm, fire_load, wait_load, need_src=False)

    hop(x0_hbm, ei0_hbm)
    ea_phase()
    hop(x1_hbm, ei1_hbm)
    hop(x2_hbm, ei2_hbm)

    plsc.subcore_barrier()

    def write_body(j, carry):
      blk = s + j * NS
      pltpu.sync_copy(acc.at[pl.ds(blk * BROWS, BROWS), :],
                      out_hbm.at[c, pl.ds(blk * BROWS, BROWS), :])
      return carry
    lax.fori_loop(0, (wblocks - s + NS - 1) // NS, write_body, 0)

  return sc_kernel


def _mlp_body(p_ref, x0_ref, w1_ref, b1_ref, g1_ref, be1_ref,
              w2_ref, b2_ref, g2_ref, be2_ref, o_ref):
  r = x0_ref[...] + p_ref[0] + p_ref[1]
  h = jnp.dot(r, w1_ref[...], preferred_element_type=jnp.float32) + b1_ref[...]
  mu = jnp.mean(h, axis=0, keepdims=True)
  var = jnp.mean(jnp.square(h - mu), axis=0, keepdims=True)
  h = jnp.maximum((h - mu) * lax.rsqrt(var + 1e-5) * g1_ref[...] + be1_ref[...], 0.0)
  h = jnp.dot(h, w2_ref[...], preferred_element_type=jnp.float32) + b2_ref[...]
  mu = jnp.mean(h, axis=0, keepdims=True)
  var = jnp.mean(jnp.square(h - mu), axis=0, keepdims=True)
  o_ref[...] = jnp.maximum((h - mu) * lax.rsqrt(var + 1e-5) * g2_ref[...] + be2_ref[...], 0.0)


def _pad_indices(ei, epad, n_dummy):
  e = ei.shape[1]
  pad = epad - e
  src = jnp.concatenate([ei[0], jnp.zeros((pad,), jnp.int32)])
  dst = jnp.concatenate([ei[1], jnp.full((pad,), n_dummy, jnp.int32)])
  return jnp.concatenate([src, dst])


def kernel(x0, x1, x2, edge_attr, W1, b1, g1, be1, W2, b2, g2, be2,
           edge_index0, edge_index1, edge_index2):
  n, d = x0.shape
  e = edge_index0.shape[1]
  assert n % BROWS == 0
  nchunks = -(-e // CHUNK)
  nchunks = -(-nchunks // (2 * NW)) * (2 * NW)
  trip = nchunks // NW
  epad = nchunks * CHUNK
  n_pad = n + 8

  ei0 = _pad_indices(edge_index0, epad, n)
  ei1 = _pad_indices(edge_index1, epad, n)
  ei2 = _pad_indices(edge_index2, epad, n)

  parts = _sc_hops(e, n, n_pad, d, trip)(
      x0, x1, x2, edge_attr, ei0, ei1, ei2)

  out = pl.pallas_call(
      _mlp_body,
      out_shape=jax.ShapeDtypeStruct((n, d), jnp.float32),
  )(parts, x0, W1.T, b1.reshape(1, d), g1.reshape(1, d), be1.reshape(1, d),
    W2.T, b2.reshape(1, d), g2.reshape(1, d), be2.reshape(1, d))
  return out
